# Initial kernel scaffold; baseline (speedup 1.0000x reference)
import jax
import jax.numpy as jnp
from jax.experimental import pallas as pl


def kernel(features, lengths, table, init_state):
    raise NotImplementedError("write your pallas kernel here")


# speedup vs baseline: 1.0618x; 1.0618x over previous
import functools

import jax
import jax.numpy as jnp
from jax import lax
from jax.experimental import pallas as pl
from jax.experimental.pallas import tpu as pltpu
from jax.experimental.pallas import tpu_sc as plsc

_NC = 2
_NS = 16
_NW = _NC * _NS

_BG = 4
_SG = 8
_IDX_CHUNK = 128


def _sc_gather(features, table):
    S, B = features.shape
    V, D = table.shape
    s_per = S // _SG
    b_per = B // _BG
    n_ch = b_per // _IDX_CHUNK

    mesh = plsc.VectorSubcoreMesh(core_axis_name="c", subcore_axis_name="s")

    @functools.partial(
        pl.kernel,
        out_type=jax.ShapeDtypeStruct((B, S, D), table.dtype),
        mesh=mesh,
        scratch_types=[
            pltpu.VMEM((s_per, b_per), jnp.int32),
            pltpu.VMEM((b_per, D), jnp.float32),
            pltpu.SemaphoreType.DMA,
        ],
        compiler_params=pltpu.CompilerParams(use_tc_tiling_on_sc=False),
    )
    def gather_kernel(feat_hbm, table_hbm, out_hbm, idx_v, rows_v, gsem):
        wid = lax.axis_index("c") * _NS + lax.axis_index("s")
        bg = wid // _SG
        sg = wid % _SG
        s0 = sg * s_per
        b0 = bg * b_per

        pltpu.sync_copy(
            feat_hbm.at[pl.ds(s0, s_per), pl.ds(b0, b_per)], idx_v
        )

        def step(s_l, carry):
            descs = [
                pltpu.async_copy(
                    table_hbm.at[idx_v.at[s_l, pl.ds(j * _IDX_CHUNK, _IDX_CHUNK)]],
                    rows_v.at[pl.ds(j * _IDX_CHUNK, _IDX_CHUNK)],
                    gsem,
                )
                for j in range(n_ch)
            ]
            for d in descs:
                d.wait()
            pltpu.sync_copy(rows_v, out_hbm.at[pl.ds(b0, b_per), s0 + s_l])
            return carry

        lax.fori_loop(0, s_per, step, None)

    return gather_kernel(features, table)


def _tc_tile_init(init_state, batch):
    L, _, H = init_state.shape
    blk = 512
    grid = (batch // blk,)

    def tile_kernel(init_ref, out_ref):
        out_ref[...] = jnp.broadcast_to(init_ref[...], out_ref.shape)

    return pl.pallas_call(
        tile_kernel,
        grid=grid,
        in_specs=[pl.BlockSpec((L, 1, H), lambda i: (0, 0, 0))],
        out_specs=pl.BlockSpec((L, blk, H), lambda i: (0, i, 0)),
        out_shape=jax.ShapeDtypeStruct((L, batch, H), init_state.dtype),
    )(init_state)


def kernel(features, lengths, table, init_state):
    del lengths
    emb = _sc_gather(features, table)
    init = _tc_tile_init(init_state, features.shape[1])
    return (emb, init)


# speedup vs baseline: 1.3834x; 1.3030x over previous
import functools

import jax
import jax.numpy as jnp
from jax import lax
from jax.experimental import pallas as pl
from jax.experimental.pallas import tpu as pltpu
from jax.experimental.pallas import tpu_sc as plsc

_NC = 2
_NS = 16
_NW = _NC * _NS
_BG = 4
_SG = 8
_IDX_CHUNK = 128
_VBLK = 8192


def _tc_pack_table(table_t):
    D, V = table_t.shape
    g = 128 // D
    v_blk = _VBLK
    rows_blk = v_blk // g
    grid = (pl.cdiv(V, v_blk),)
    n_rows = grid[0] * rows_blk

    def pack_kernel(in_ref, out_ref):
        a = in_ref[...]
        for q in range(g):
            out_ref[:, q * D:(q + 1) * D] = a[:, q * rows_blk:(q + 1) * rows_blk].T

    return pl.pallas_call(
        pack_kernel,
        grid=grid,
        in_specs=[pl.BlockSpec((D, v_blk), lambda i: (0, i))],
        out_specs=pl.BlockSpec((rows_blk, 128), lambda i: (i, 0)),
        out_shape=jax.ShapeDtypeStruct((n_rows, 128), table_t.dtype),
    )(table_t)


def _sc_gather(features, table_lin):
    S, B = features.shape
    V, D = table_lin.shape
    s_per = S // _SG
    b_per = B // _BG
    n_ch = b_per // _IDX_CHUNK

    mesh = plsc.VectorSubcoreMesh(core_axis_name="c", subcore_axis_name="s")

    @functools.partial(
        pl.kernel,
        out_type=jax.ShapeDtypeStruct((B, S, D), table_lin.dtype),
        mesh=mesh,
        scratch_types=[
            pltpu.VMEM((s_per, b_per), jnp.int32),
            pltpu.VMEM((b_per, D), jnp.float32),
            pltpu.SemaphoreType.DMA,
        ],
        compiler_params=pltpu.CompilerParams(use_tc_tiling_on_sc=False),
    )
    def gather_kernel(feat_hbm, table_hbm, out_hbm, idx_v, rows_v, gsem):
        wid = lax.axis_index("c") * _NS + lax.axis_index("s")
        bg = wid // _SG
        sg = wid % _SG
        s0 = sg * s_per
        b0 = bg * b_per

        pltpu.sync_copy(
            feat_hbm.at[pl.ds(s0, s_per), pl.ds(b0, b_per)], idx_v
        )

        n_vec = b_per // 16
        def remap(t, carry):
            s_l = t // n_vec
            i = t % n_vec
            v = idx_v[s_l, pl.ds(i * 16, 16)]
            k = (v & -8192) | ((v & 2047) << 2) | ((v >> 11) & 3)
            idx_v[s_l, pl.ds(i * 16, 16)] = k
            return carry

        lax.fori_loop(0, s_per * n_vec, remap, None)

        def step(s_l, carry):
            descs = [
                pltpu.async_copy(
                    table_hbm.at[idx_v.at[s_l, pl.ds(j * _IDX_CHUNK, _IDX_CHUNK)]],
                    rows_v.at[pl.ds(j * _IDX_CHUNK, _IDX_CHUNK)],
                    gsem,
                )
                for j in range(n_ch)
            ]
            for d in descs:
                d.wait()
            pltpu.sync_copy(rows_v, out_hbm.at[pl.ds(b0, b_per), s0 + s_l])
            return carry

        lax.fori_loop(0, s_per, step, None)

    return gather_kernel(features, table_lin)


def _tc_tile_init(init_state, batch):
    L, _, H = init_state.shape
    blk = 512
    grid = (batch // blk,)

    def tile_kernel(init_ref, out_ref):
        out_ref[...] = jnp.broadcast_to(init_ref[...], out_ref.shape)

    return pl.pallas_call(
        tile_kernel,
        grid=grid,
        in_specs=[pl.BlockSpec((L, 1, H), lambda i: (0, 0, 0))],
        out_specs=pl.BlockSpec((L, blk, H), lambda i: (0, i, 0)),
        out_shape=jax.ShapeDtypeStruct((L, batch, H), init_state.dtype),
    )(init_state)


def kernel(features, lengths, table, init_state):
    del lengths
    V, D = table.shape
    t_pack = _tc_pack_table(jnp.transpose(table))
    table_lin = jnp.reshape(t_pack, (t_pack.shape[0] * (128 // D), D))
    emb = _sc_gather(features, table_lin)
    init = _tc_tile_init(init_state, features.shape[1])
    return (emb, init)
